# baseline (speedup 1.0000x reference)
import functools

import jax
import jax.numpy as jnp
from jax import lax
from jax.experimental import pallas as pl
from jax.experimental.pallas import tpu as pltpu
from jax.experimental.pallas import tpu_sc as plsc

_T = 16384
_D = 2048
_LANES = 16
_NC = 2
_NS = 16
_NW = _NC * _NS
_ROWS_PER_W = _T // _NW
_CHUNK = 8
_NCHUNK = _ROWS_PER_W // _CHUNK


def _make_kernel():
  mesh = plsc.VectorSubcoreMesh(core_axis_name="c", subcore_axis_name="s")

  @functools.partial(
      pl.kernel,
      mesh=mesh,
      out_type=jax.ShapeDtypeStruct((_T, _D), jnp.float32),
      scratch_types=[
          pltpu.VMEM((_CHUNK, _D), jnp.float32),
          pltpu.VMEM((_CHUNK, _D), jnp.float32),
          pltpu.VMEM((_CHUNK, _D), jnp.float32),
          pltpu.VMEM((_CHUNK, _D), jnp.float32),
          pltpu.VMEM((1, _D), jnp.float32),
          pltpu.VMEM((1,), jnp.int32),
          pltpu.SemaphoreType.DMA,
          pltpu.SemaphoreType.DMA,
          pltpu.SemaphoreType.DMA,
          pltpu.SemaphoreType.DMA,
      ],
  )
  def add_embed(x_hbm, idx_hbm, emb_hbm, out_hbm,
                in0, in1, ou0, ou1, emb_v, idx_v, si0, si1, so0, so1):
    wid = lax.axis_index("s") * _NC + lax.axis_index("c")
    base = wid * _ROWS_PER_W

    pltpu.sync_copy(idx_hbm, idx_v)
    pltpu.async_copy(emb_hbm.at[idx_v], emb_v, so0).wait()

    inbufs = (in0, in1)
    outbufs = (ou0, ou1)
    isems = (si0, si1)
    osems = (so0, so1)

    def start_in(ch, b):
      pltpu.async_copy(
          x_hbm.at[pl.ds(base + ch * _CHUNK, _CHUNK)], inbufs[b], isems[b])

    start_in(0, 0)
    start_in(1, 1)

    def outer(i, _):
      c = i * 2
      for b in range(2):
        ch = c + b
        pltpu.make_async_copy(
            x_hbm.at[pl.ds(0, _CHUNK)], inbufs[b], isems[b]).wait()

        @pl.when(ch >= 2)
        def _():
          pltpu.make_async_copy(
              outbufs[b], out_hbm.at[pl.ds(0, _CHUNK)], osems[b]).wait()

        ib = inbufs[b]
        ob = outbufs[b]

        def col_body(j2, _):
          for u in range(2):
            col = pl.multiple_of((j2 * 2 + u) * _LANES, _LANES)
            ev = emb_v[0, pl.ds(col, _LANES)]
            for r in range(_CHUNK):
              ob[r, pl.ds(col, _LANES)] = ib[r, pl.ds(col, _LANES)] + ev
          return 0

        lax.fori_loop(0, _D // _LANES // 2, col_body, 0)

        pltpu.async_copy(
            outbufs[b], out_hbm.at[pl.ds(base + ch * _CHUNK, _CHUNK)],
            osems[b])

        @pl.when(ch + 2 < _NCHUNK)
        def _():
          start_in(ch + 2, b)

      return 0

    lax.fori_loop(0, _NCHUNK // 2, outer, 0)

    for b in range(2):
      pltpu.make_async_copy(
          outbufs[b], out_hbm.at[pl.ds(0, _CHUNK)], osems[b]).wait()

  return add_embed


_add_embed_call = _make_kernel()


@jax.jit
def kernel(input_features, modality_indices, embedding_weight):
  out = _add_embed_call(
      input_features, modality_indices.astype(jnp.int32), embedding_weight
  )
  return out[None]


# speedup vs baseline: 1.4247x; 1.4247x over previous
import functools

import jax
import jax.numpy as jnp
from jax import lax
from jax.experimental import pallas as pl
from jax.experimental.pallas import tpu as pltpu
from jax.experimental.pallas import tpu_sc as plsc

_T = 16384
_D = 2048
_LANES = 16
_NC = 2
_NS = 16
_NW = _NC * _NS
_ROWS_PER_W = _T // _NW
_CHUNK = 4
_NCHUNK = _ROWS_PER_W // _CHUNK
_NBUF = 8
_K = 4


def _make_kernel():
  mesh = plsc.VectorSubcoreMesh(core_axis_name="c", subcore_axis_name="s")

  @functools.partial(
      pl.kernel,
      mesh=mesh,
      out_type=jax.ShapeDtypeStruct((_T, _D), jnp.float32),
      scratch_types=(
          [pltpu.VMEM((_CHUNK, _D), jnp.float32)] * _NBUF
          + [pltpu.VMEM((1, _D), jnp.float32), pltpu.VMEM((1,), jnp.int32)]
          + [pltpu.SemaphoreType.DMA] * (2 * _NBUF)
      ),
  )
  def add_embed(x_hbm, idx_hbm, emb_hbm, out_hbm, *refs):
    bufs = refs[:_NBUF]
    emb_v = refs[_NBUF]
    idx_v = refs[_NBUF + 1]
    isems = refs[_NBUF + 2:_NBUF + 2 + _NBUF]
    osems = refs[_NBUF + 2 + _NBUF:]

    wid = lax.axis_index("s") * _NC + lax.axis_index("c")
    base = wid * _ROWS_PER_W

    pltpu.sync_copy(idx_hbm, idx_v)
    pltpu.async_copy(emb_hbm.at[idx_v], emb_v, osems[0]).wait()

    def start_in(ch, b):
      pltpu.async_copy(
          x_hbm.at[pl.ds(base + ch * _CHUNK, _CHUNK)], bufs[b], isems[b])

    for ch in range(_K):
      start_in(ch, ch % _NBUF)

    def outer(i, _):
      c = i * _NBUF
      for b in range(_NBUF):
        ch = c + b

        t = ch + _K
        bt = (b + _K) % _NBUF

        @pl.when(t < _NCHUNK)
        def _():
          @pl.when(t >= _NBUF)
          def _():
            pltpu.make_async_copy(
                bufs[bt], out_hbm.at[pl.ds(0, _CHUNK)], osems[bt]).wait()

          start_in(t, bt)

        pltpu.make_async_copy(
            x_hbm.at[pl.ds(0, _CHUNK)], bufs[b], isems[b]).wait()

        def col_body(j, _):
          col = pl.multiple_of(j * _LANES, _LANES)
          ev = emb_v[0, pl.ds(col, _LANES)]
          for r in range(_CHUNK):
            plsc.addupdate(bufs[b].at[r, pl.ds(col, _LANES)], ev)
          return 0

        lax.fori_loop(0, _D // _LANES, col_body, 0)

        pltpu.async_copy(
            bufs[b], out_hbm.at[pl.ds(base + ch * _CHUNK, _CHUNK)], osems[b])

      return 0

    lax.fori_loop(0, _NCHUNK // _NBUF, outer, 0)

    for b in range(_NBUF):
      pltpu.make_async_copy(
          bufs[b], out_hbm.at[pl.ds(0, _CHUNK)], osems[b]).wait()

  return add_embed


_add_embed_call = _make_kernel()


@jax.jit
def kernel(input_features, modality_indices, embedding_weight):
  out = _add_embed_call(
      input_features, modality_indices.astype(jnp.int32), embedding_weight
  )
  return out[None]


# speedup vs baseline: 1.4824x; 1.0405x over previous
import functools

import jax
import jax.numpy as jnp
from jax import lax
from jax.experimental import pallas as pl
from jax.experimental.pallas import tpu as pltpu
from jax.experimental.pallas import tpu_sc as plsc

_T = 16384
_D = 2048
_LANES = 16
_NC = 2
_NS = 16
_NW = _NC * _NS
_ROWS_PER_W = _T // _NW
_CHUNK = 8
_NCHUNK = _ROWS_PER_W // _CHUNK
_NBUF = 4
_K = 2


def _make_kernel():
  mesh = plsc.VectorSubcoreMesh(core_axis_name="c", subcore_axis_name="s")

  @functools.partial(
      pl.kernel,
      mesh=mesh,
      out_type=jax.ShapeDtypeStruct((_T, _D), jnp.float32),
      scratch_types=(
          [pltpu.VMEM((_CHUNK, _D), jnp.float32)] * _NBUF
          + [pltpu.VMEM((1, _D), jnp.float32), pltpu.VMEM((1,), jnp.int32)]
          + [pltpu.SemaphoreType.DMA] * (2 * _NBUF)
      ),
  )
  def add_embed(x_hbm, idx_hbm, emb_hbm, out_hbm, *refs):
    bufs = refs[:_NBUF]
    emb_v = refs[_NBUF]
    idx_v = refs[_NBUF + 1]
    isems = refs[_NBUF + 2:_NBUF + 2 + _NBUF]
    osems = refs[_NBUF + 2 + _NBUF:]

    wid = lax.axis_index("s") * _NC + lax.axis_index("c")
    base = wid * _ROWS_PER_W

    pltpu.sync_copy(idx_hbm, idx_v)
    pltpu.async_copy(emb_hbm.at[idx_v], emb_v, osems[0]).wait()

    def start_in(ch, b):
      pltpu.async_copy(
          x_hbm.at[pl.ds(base + ch * _CHUNK, _CHUNK)], bufs[b], isems[b])

    for ch in range(_K):
      start_in(ch, ch % _NBUF)

    def outer(i, _):
      c = i * _NBUF
      for b in range(_NBUF):
        ch = c + b

        t = ch + _K
        bt = (b + _K) % _NBUF

        @pl.when(t < _NCHUNK)
        def _():
          @pl.when(t >= _NBUF)
          def _():
            pltpu.make_async_copy(
                bufs[bt], out_hbm.at[pl.ds(0, _CHUNK)], osems[bt]).wait()

          start_in(t, bt)

        pltpu.make_async_copy(
            x_hbm.at[pl.ds(0, _CHUNK)], bufs[b], isems[b]).wait()

        def col_body(j, _):
          col = pl.multiple_of(j * _LANES, _LANES)
          ev = emb_v[0, pl.ds(col, _LANES)]
          for r in range(_CHUNK):
            plsc.addupdate(bufs[b].at[r, pl.ds(col, _LANES)], ev)
          return 0

        lax.fori_loop(0, _D // _LANES, col_body, 0)

        pltpu.async_copy(
            bufs[b], out_hbm.at[pl.ds(base + ch * _CHUNK, _CHUNK)], osems[b])

      return 0

    lax.fori_loop(0, _NCHUNK // _NBUF, outer, 0)

    for b in range(_NBUF):
      pltpu.make_async_copy(
          bufs[b], out_hbm.at[pl.ds(0, _CHUNK)], osems[b]).wait()

  return add_embed


_add_embed_call = _make_kernel()


@jax.jit
def kernel(input_features, modality_indices, embedding_weight):
  out = _add_embed_call(
      input_features, modality_indices.astype(jnp.int32), embedding_weight
  )
  return out[None]
